# baseline (speedup 1.0000x reference)
import functools

import jax
import jax.numpy as jnp
from jax import lax
from jax.experimental import pallas as pl
from jax.experimental.pallas import tpu as pltpu
from jax.experimental.pallas import tpu_sc as plsc

_TIMESTEPS = 1000
_N = 16384
_NC = 1
_NS = 16
_NW = _NC * _NS
_CHUNK = _N // _NW
_LANES = 16
_G = 1001

_mesh = plsc.VectorSubcoreMesh(
    core_axis_name="c", subcore_axis_name="s", num_cores=_NC
)


@functools.partial(
    pl.kernel,
    mesh=_mesh,
    out_type=jax.ShapeDtypeStruct((_N,), jnp.float32),
    compiler_params=pltpu.CompilerParams(
        needs_layout_passes=False, use_tc_tiling_on_sc=False
    ),
    scratch_types=[
        pltpu.VMEM((_G,), jnp.float32),
        pltpu.VMEM_SHARED((_G,), jnp.float32),
        pltpu.VMEM((_CHUNK,), jnp.float32),
        pltpu.VMEM((_CHUNK,), jnp.int32),
        pltpu.VMEM((_CHUNK,), jnp.float32),
        pltpu.SemaphoreType.DMA,
        pltpu.SemaphoreType.DMA,
        pltpu.SemaphoreType.DMA,
    ],
)
def _sc_lookup(t_hbm, gamma_hbm, out_hbm, gamma_v, gamma_sh, t_v, i_v, o_v,
               sem_g, sem_t, sem_o):
    sid = lax.axis_index("s")
    base = sid * _CHUNK
    cp_t = pltpu.async_copy(t_hbm.at[pl.ds(base, _CHUNK)], t_v, sem_t)

    @pl.when(sid == 0)
    def _():
        pltpu.sync_copy(gamma_hbm, gamma_sh)

    plsc.subcore_barrier()
    cp_g = pltpu.async_copy(gamma_sh, gamma_v, sem_g)
    cp_t.wait()

    magic_f = jnp.float32(8388608.0)
    magic_i = jnp.int32(0x4B000000)

    def index_body(i, carry):
        y = t_v[pl.ds(i * _LANES, _LANES)] * jnp.float32(_TIMESTEPS) + magic_f
        i_v[pl.ds(i * _LANES, _LANES)] = plsc.bitcast(y, jnp.int32) - magic_i
        return carry

    def gather_body(i, carry):
        idx = i_v[pl.ds(i * _LANES, _LANES)]
        o_v[pl.ds(i * _LANES, _LANES)] = plsc.load_gather(gamma_v, [idx])
        return carry

    n_vregs = _CHUNK // _LANES
    lax.fori_loop(0, n_vregs, index_body, 0, unroll=8)
    cp_g.wait()

    half = _CHUNK // 2
    half_vregs = half // _LANES
    lax.fori_loop(0, half_vregs, gather_body, 0, unroll=8)
    cp_o1 = pltpu.async_copy(
        o_v.at[pl.ds(0, half)], out_hbm.at[pl.ds(base, half)], sem_o
    )
    lax.fori_loop(half_vregs, 2 * half_vregs, gather_body, 0, unroll=8)
    cp_o2 = pltpu.async_copy(
        o_v.at[pl.ds(half, half)], out_hbm.at[pl.ds(base + half, half)], sem_o
    )
    cp_o1.wait()
    cp_o2.wait()


def kernel(t, gamma):
    out = _sc_lookup(t.reshape(_N), gamma)
    return out.reshape(t.shape)


# speedup vs baseline: 1.0011x; 1.0011x over previous
import functools

import jax
import jax.numpy as jnp
from jax import lax
from jax.experimental import pallas as pl
from jax.experimental.pallas import tpu as pltpu
from jax.experimental.pallas import tpu_sc as plsc

_TIMESTEPS = 1000
_N = 16384
_NC = 1
_NS = 16
_NW = _NC * _NS
_CHUNK = _N // _NW
_LANES = 16
_G = 1001

_mesh = plsc.VectorSubcoreMesh(
    core_axis_name="c", subcore_axis_name="s", num_cores=_NC
)


@functools.partial(
    pl.kernel,
    mesh=_mesh,
    out_type=jax.ShapeDtypeStruct((_N,), jnp.float32),
    compiler_params=pltpu.CompilerParams(
        needs_layout_passes=False,
        use_tc_tiling_on_sc=False,
        disable_bounds_checks=True,
        disable_semaphore_checks=True,
    ),
    scratch_types=[
        pltpu.VMEM((_G,), jnp.float32),
        pltpu.VMEM_SHARED((_G,), jnp.float32),
        pltpu.VMEM((_CHUNK,), jnp.float32),
        pltpu.VMEM((_CHUNK,), jnp.float32),
        pltpu.SemaphoreType.DMA,
        pltpu.SemaphoreType.DMA,
        pltpu.SemaphoreType.DMA,
    ],
)
def _sc_lookup(t_hbm, gamma_hbm, out_hbm, gamma_v, gamma_sh, t_v, o_v, sem_g,
               sem_t, sem_o):
    sid = lax.axis_index("s")
    base = sid * _CHUNK
    cp_t = pltpu.async_copy(t_hbm.at[pl.ds(base, _CHUNK)], t_v, sem_t)

    @pl.when(sid == 0)
    def _():
        pltpu.sync_copy(gamma_hbm, gamma_sh)

    plsc.subcore_barrier()
    cp_g = pltpu.async_copy(gamma_sh, gamma_v, sem_g)
    cp_g.wait()
    cp_t.wait()

    magic_f = jnp.float32(8388608.0)
    magic_i = jnp.int32(0x4B000000)

    def body(i, carry):
        y = t_v[pl.ds(i * _LANES, _LANES)] * jnp.float32(_TIMESTEPS) + magic_f
        idx = plsc.bitcast(y, jnp.int32) - magic_i
        o_v[pl.ds(i * _LANES, _LANES)] = plsc.load_gather(gamma_v, [idx])
        return carry

    half = _CHUNK // 2
    half_vregs = half // _LANES
    lax.fori_loop(0, half_vregs, body, 0, unroll=8)
    cp_o1 = pltpu.async_copy(
        o_v.at[pl.ds(0, half)], out_hbm.at[pl.ds(base, half)], sem_o
    )
    lax.fori_loop(half_vregs, 2 * half_vregs, body, 0, unroll=8)
    cp_o2 = pltpu.async_copy(
        o_v.at[pl.ds(half, half)], out_hbm.at[pl.ds(base + half, half)], sem_o
    )
    cp_o1.wait()
    cp_o2.wait()


def kernel(t, gamma):
    out = _sc_lookup(t.reshape(_N), gamma)
    return out.reshape(t.shape)


# speedup vs baseline: 1.0019x; 1.0008x over previous
import functools

import jax
import jax.numpy as jnp
from jax import lax
from jax.experimental import pallas as pl
from jax.experimental.pallas import tpu as pltpu
from jax.experimental.pallas import tpu_sc as plsc

_TIMESTEPS = 1000
_N = 16384
_NS = 16
_CHUNK = _N // _NS
_LANES = 16
_G = 1001

_mesh = plsc.VectorSubcoreMesh(
    core_axis_name="c", subcore_axis_name="s", num_cores=1
)


@functools.partial(
    pl.kernel,
    mesh=_mesh,
    out_type=jax.ShapeDtypeStruct((_N,), jnp.float32),
    compiler_params=pltpu.CompilerParams(
        needs_layout_passes=False, use_tc_tiling_on_sc=False
    ),
    scratch_types=[
        pltpu.VMEM((_G,), jnp.float32),
        pltpu.VMEM_SHARED((_G,), jnp.float32),
        pltpu.VMEM((_CHUNK,), jnp.float32),
        pltpu.VMEM((_CHUNK,), jnp.float32),
        pltpu.SemaphoreType.DMA,
        pltpu.SemaphoreType.DMA,
        pltpu.SemaphoreType.DMA,
    ],
)
def _sc_lookup(t_hbm, gamma_hbm, out_hbm, gamma_v, gamma_sh, t_v, o_v, sem_g,
               sem_t, sem_o):
    sid = lax.axis_index("s")
    base = sid * _CHUNK
    cp_t = pltpu.async_copy(t_hbm.at[pl.ds(base, _CHUNK)], t_v, sem_t)

    @pl.when(sid == 0)
    def _():
        pltpu.sync_copy(gamma_hbm, gamma_sh)

    plsc.subcore_barrier()
    cp_g = pltpu.async_copy(gamma_sh, gamma_v, sem_g)
    cp_g.wait()
    cp_t.wait()

    magic_f = jnp.float32(8388608.0)
    magic_i = jnp.int32(0x4B000000)

    def body(i, carry):
        y = t_v[pl.ds(i * _LANES, _LANES)] * jnp.float32(_TIMESTEPS) + magic_f
        idx = plsc.bitcast(y, jnp.int32) - magic_i
        o_v[pl.ds(i * _LANES, _LANES)] = plsc.load_gather(gamma_v, [idx])
        return carry

    half = _CHUNK // 2
    half_vregs = half // _LANES
    lax.fori_loop(0, half_vregs, body, 0, unroll=8)
    cp_o1 = pltpu.async_copy(
        o_v.at[pl.ds(0, half)], out_hbm.at[pl.ds(base, half)], sem_o
    )
    lax.fori_loop(half_vregs, 2 * half_vregs, body, 0, unroll=8)
    cp_o2 = pltpu.async_copy(
        o_v.at[pl.ds(half, half)], out_hbm.at[pl.ds(base + half, half)], sem_o
    )
    cp_o1.wait()
    cp_o2.wait()


def kernel(t, gamma):
    out = _sc_lookup(t.reshape(_N), gamma)
    return out.reshape(t.shape)
